# Initial kernel scaffold; baseline (speedup 1.0000x reference)
import jax
import jax.numpy as jnp
from jax.experimental import pallas as pl


def kernel(log_probs, targets):
    raise NotImplementedError("write your pallas kernel here")


# speedup vs baseline: 8.6657x; 8.6657x over previous
import math

import jax
import jax.numpy as jnp
from jax.experimental import pallas as pl

_N = 4096
_VOCAB = 32000
_IGNORE = 0
_EPS = 0.1
_U = _EPS / (_VOCAB - 2)
_C = (1.0 - _EPS) * math.log(1.0 - _EPS) + (_VOCAB - 2) * _U * math.log(_U)

_ROW_BLOCK = 128


def _loss_kernel(tgt_ref, lp_ref, out_ref):
    i = pl.program_id(0)
    tile = lp_ref[...]
    tgt = tgt_ref[0, 0, :]
    cols = jax.lax.broadcasted_iota(jnp.int32, tile.shape, 1)
    rowsum = jnp.sum(tile, axis=1)
    lp_t = jnp.sum(jnp.where(cols == tgt[:, None], tile, 0.0), axis=1)
    lp_0 = tile[:, 0]
    cross = _U * (rowsum - lp_0) + (1.0 - _EPS - _U) * lp_t
    contrib = jnp.where(tgt == _IGNORE, 0.0, _C - cross)
    partial = jnp.sum(contrib.reshape(1, -1), axis=1, keepdims=True)

    @pl.when(i == 0)
    def _():
        out_ref[...] = jnp.zeros_like(out_ref)

    out_ref[...] += partial


def kernel(log_probs, targets):
    n, v = log_probs.shape
    r = _ROW_BLOCK
    nb = n // r
    tgt3 = targets.reshape(nb, 1, r)
    out = pl.pallas_call(
        _loss_kernel,
        grid=(nb,),
        in_specs=[
            pl.BlockSpec((1, 1, r), lambda i: (i, 0, 0)),
            pl.BlockSpec((r, v), lambda i: (i, 0)),
        ],
        out_specs=pl.BlockSpec((1, 1), lambda i: (0, 0)),
        out_shape=jax.ShapeDtypeStruct((1, 1), jnp.float32),
    )(tgt3, log_probs)
    return out[0, 0]
